# Initial kernel scaffold; baseline (speedup 1.0000x reference)
import jax
import jax.numpy as jnp
from jax.experimental import pallas as pl


def kernel(x, conv_w, conv_b, bn_gamma, bn_beta):
    raise NotImplementedError("write your pallas kernel here")


# speedup vs baseline: 18.5816x; 18.5816x over previous
import functools

import jax
import jax.numpy as jnp
from jax import lax
from jax.experimental import pallas as pl
from jax.experimental.pallas import tpu as pltpu

B, C_IN, C_OUT, N, K = 4, 64, 128, 2048, 16
TN = 256
NEG_BIG = 3.0e38


def _knn_max_body(rows_ref, full_ref, xm_ref):
    rows = rows_ref[0]
    full = full_ref[0]
    rn_rows = jnp.sum(rows * rows, axis=1, keepdims=True)
    rn_all = jnp.sum(full * full, axis=1)[None, :]
    cross = lax.dot_general(
        rows, full, dimension_numbers=(((1,), (1,)), ((), ())),
        preferred_element_type=jnp.float32)
    d = rn_rows + rn_all - 2.0 * cross

    xm = jnp.full((TN, C_IN), -NEG_BIG, dtype=jnp.float32)
    for _ in range(K):
        v = jnp.min(d, axis=1, keepdims=True)
        eq = d == v
        oh = eq.astype(jnp.float32)
        gathered = lax.dot_general(
            oh, full, dimension_numbers=(((1,), (0,)), ((), ())),
            preferred_element_type=jnp.float32)
        xm = jnp.maximum(xm, gathered)
        d = jnp.where(eq, NEG_BIG, d)
    xm_ref[0] = xm


def _conv_stats_body(xm_ref, w_ref, b_ref, y_ref, sums_ref):
    step = pl.program_id(0) * pl.num_programs(1) + pl.program_id(1)
    xm = xm_ref[0]
    w = w_ref[...]
    y = lax.dot_general(
        xm, w, dimension_numbers=(((1,), (1,)), ((), ())),
        preferred_element_type=jnp.float32) + b_ref[...]
    y_ref[0] = y

    @pl.when(step == 0)
    def _():
        sums_ref[...] = jnp.zeros_like(sums_ref)

    s1 = jnp.sum(y, axis=0, keepdims=True)
    s2 = jnp.sum(y * y, axis=0, keepdims=True)
    sums_ref[0:1, :] += s1
    sums_ref[1:2, :] += s2


def _bn_relu_body(y_ref, sums_ref, g_ref, bt_ref, out_ref):
    y = y_ref[0]
    cnt = float(B * N)
    mean = sums_ref[0:1, :] / cnt
    var = sums_ref[1:2, :] / cnt - mean * mean
    scale = g_ref[...] / jnp.sqrt(var + 1e-5)
    shift = bt_ref[...] - mean * scale
    r = jnp.maximum(y * scale + shift, 0.0)
    out_ref[0] = r.T


def kernel(x, conv_w, conv_b, bn_gamma, bn_beta):
    xt = jnp.transpose(x, (0, 2, 1))
    w = conv_w[:, :, 0]

    xm = pl.pallas_call(
        _knn_max_body,
        grid=(B, N // TN),
        in_specs=[
            pl.BlockSpec((1, TN, C_IN), lambda b, i: (b, i, 0)),
            pl.BlockSpec((1, N, C_IN), lambda b, i: (b, 0, 0)),
        ],
        out_specs=pl.BlockSpec((1, TN, C_IN), lambda b, i: (b, i, 0)),
        out_shape=jax.ShapeDtypeStruct((B, N, C_IN), jnp.float32),
    )(xt, xt)

    y, sums = pl.pallas_call(
        _conv_stats_body,
        grid=(B, N // TN),
        in_specs=[
            pl.BlockSpec((1, TN, C_IN), lambda b, i: (b, i, 0)),
            pl.BlockSpec((C_OUT, C_IN), lambda b, i: (0, 0)),
            pl.BlockSpec((1, C_OUT), lambda b, i: (0, 0)),
        ],
        out_specs=[
            pl.BlockSpec((1, TN, C_OUT), lambda b, i: (b, i, 0)),
            pl.BlockSpec((8, C_OUT), lambda b, i: (0, 0)),
        ],
        out_shape=[
            jax.ShapeDtypeStruct((B, N, C_OUT), jnp.float32),
            jax.ShapeDtypeStruct((8, C_OUT), jnp.float32),
        ],
    )(xm, w, conv_b[None, :])

    out = pl.pallas_call(
        _bn_relu_body,
        grid=(B, N // TN),
        in_specs=[
            pl.BlockSpec((1, TN, C_OUT), lambda b, i: (b, i, 0)),
            pl.BlockSpec((8, C_OUT), lambda b, i: (0, 0)),
            pl.BlockSpec((1, C_OUT), lambda b, i: (0, 0)),
            pl.BlockSpec((1, C_OUT), lambda b, i: (0, 0)),
        ],
        out_specs=pl.BlockSpec((1, C_OUT, TN), lambda b, i: (b, 0, i)),
        out_shape=jax.ShapeDtypeStruct((B, C_OUT, N), jnp.float32),
    )(y, sums, bn_gamma[None, :], bn_beta[None, :])

    return out
